# baseline (speedup 1.0000x reference)
import functools

import jax
import jax.numpy as jnp
from jax import lax
from jax.experimental import pallas as pl
from jax.experimental.pallas import tpu as pltpu
from jax.experimental.pallas import tpu_sc as plsc

NUM_USERS = 1000000
NUM_TEAMS = 1000
EMBED_DIM = 32
BATCH = 16384

_info = plsc.get_sparse_core_info()
_NC, _NS = _info.num_cores, _info.num_subcores
_NW = _NC * _NS
_BPW = BATCH // _NW

_TAIL_LO = (NUM_USERS // 128) * 128
_SZ0 = (_TAIL_LO // 128 // _NW) * 128
_SZ31 = _TAIL_LO - (_NW - 1) * _SZ0
_CW = 2048
_CSH = 11
_FH = 8
_NPASS = EMBED_DIM // _FH
_QCAP = 640
_NSTREAM = _QCAP // 128
_OUT_ROWS = BATCH + 2 * 128
_BCAP = 96
_NCH = 16
_NCHP = _NCH + 1
_TAILBK = _NCHP

_mesh = plsc.VectorSubcoreMesh(core_axis_name="c", subcore_axis_name="s")


def _i16(x):
    return jnp.full((16,), x, dtype=jnp.int32)


@functools.partial(
    pl.kernel,
    mesh=_mesh,
    out_type=jax.ShapeDtypeStruct((_OUT_ROWS, 128), jnp.float32),
    compiler_params=pltpu.CompilerParams(needs_layout_passes=False),
    scratch_types=[
        pltpu.VMEM((8, 128), jnp.int32),
        pltpu.VMEM((8, 128), jnp.int32),
        pltpu.VMEM((_QCAP,), jnp.int32),
        pltpu.VMEM((_NSTREAM, 128), jnp.int32),
        pltpu.VMEM((_QCAP,), jnp.int32),
        pltpu.VMEM((2, _FH, _CW), jnp.float32),
        pltpu.VMEM((EMBED_DIM, 64), jnp.float32),
        pltpu.VMEM((_QCAP, 128), jnp.float32),
        pltpu.VMEM((_TAILBK + 1, _BCAP), jnp.int32),
        pltpu.SemaphoreType.DMA,
        pltpu.SemaphoreType.DMA,
    ],
)
def _scan_lookup(uidx_hbm, tidx_hbm, utabT_hbm, ttabT_hbm, tailT_hbm, out_hbm,
                 up_v, tp_v, qu_v, qb_v, qt_v, chunk_v, tail_v, ans_v,
                 bpack_v, sem, sem2):
    wid = lax.axis_index("s") * _NC + lax.axis_index("c")
    gbase = wid * _SZ0
    lo16 = _i16(gbase)
    hi16 = jnp.where(_i16(wid) == _i16(_NW - 1),
                     _i16(_TAIL_LO), _i16(gbase + _SZ0))
    blo16 = _i16(wid * _BPW)
    bhi16 = _i16(wid * _BPW + _BPW)
    tail16 = _i16(_TAIL_LO)
    iota = lax.iota(jnp.int32, 16)

    def init_q(s, _):
        for h in range(8):
            qb_v[s, pl.ds(h * 16, 16)] = _i16(BATCH + wid * 8) + (iota & 7)
            qu_v[pl.ds(s * 128 + h * 16, 16)] = _i16(-1)
        return 0

    lax.fori_loop(0, _NSTREAM, init_q, 0)

    def piece(p8, base16):
        def group(i, b16c):
            gr = i >> 3
            gc = i & 7
            u16 = up_v[gr, pl.ds(gc * 16, 16)]
            t16 = tp_v[gr, pl.ds(gc * 16, 16)]
            b16 = _i16(p8 * 1024) + _i16(i * 16) + iota
            m_main = (u16 >= lo16) & (u16 < hi16)
            m_tail = (u16 >= tail16) & (b16 >= blo16) & (b16 < bhi16)
            m = m_main | m_tail
            pos = b16c + plsc.cumsum(m.astype(jnp.int32)) - 1
            pos = jnp.minimum(pos, _QCAP - 2)
            plsc.store_scatter(qu_v, [pos], u16, mask=m)
            plsc.store_scatter(qt_v, [pos], t16, mask=m)
            plsc.store_scatter(qb_v, [pos >> 7, pos & 127], b16, mask=m)
            return b16c + plsc.all_reduce_population_count(m)

        pltpu.sync_copy(uidx_hbm.at[pl.ds(p8 * 8, 8)], up_v)
        pltpu.sync_copy(tidx_hbm.at[pl.ds(p8 * 8, 8)], tp_v)
        return lax.fori_loop(0, 64, group, base16)

    @pl.loop(0, 16, init_carry=_i16(0))
    def base16(p8, carry):
        return piece(p8, carry)

    def init_b(bk, _):
        for h in range(_BCAP // 16):
            bpack_v[bk, pl.ds(h * 16, 16)] = _i16((_QCAP - 1) << _CSH)
        return 0

    lax.fori_loop(0, _TAILBK + 1, init_b, 0)

    @pl.loop(0, _NCH)
    def _bucket(bk):
        def bkt(qg, cnt16):
            u16 = qu_v[pl.ds(qg * 16, 16)]
            rel = u16 - lo16
            m = ((rel >> _CSH) == _i16(bk)) & (u16 >= lo16) & (u16 < hi16)
            pos = jnp.minimum(cnt16 + plsc.cumsum(m.astype(jnp.int32)) - 1,
                              _BCAP - 1)
            slot = _i16(qg * 16) + iota
            plsc.store_scatter(bpack_v, [_i16(bk), pos],
                               (rel - _i16(bk * _CW)) | (slot << _CSH), mask=m)
            return cnt16 + plsc.all_reduce_population_count(m)

        lax.fori_loop(0, _QCAP // 16, bkt, _i16(0))

    def tailbkt(qg, cnt16):
        u16 = qu_v[pl.ds(qg * 16, 16)]
        m = u16 >= tail16
        pos = jnp.minimum(cnt16 + plsc.cumsum(m.astype(jnp.int32)) - 1,
                          _BCAP - 1)
        slot = _i16(qg * 16) + iota
        plsc.store_scatter(bpack_v, [_i16(_TAILBK), pos],
                           (u16 - tail16) | (slot << _CSH), mask=m)
        return cnt16 + plsc.all_reduce_population_count(m)

    lax.fori_loop(0, _QCAP // 16, tailbkt, _i16(0))

    pltpu.sync_copy(tailT_hbm, tail_v)

    def team_pass(p):
        def grp(qg, _):
            tcol = jnp.clip(qt_v[pl.ds(qg * 16, 16)], 0, _CW - 1)
            slot = _i16(qg * 16) + iota
            for f in range(_FH):
                v16 = plsc.load_gather(chunk_v, [_i16(0), _i16(f), tcol])
                plsc.store_scatter(ans_v, [slot, _i16(EMBED_DIM + p * _FH + f)], v16)
            return 0

        pltpu.sync_copy(ttabT_hbm.at[pl.ds(p * _FH, _FH)], chunk_v.at[0])
        lax.fori_loop(0, _QCAP // 16, grp, 0)

    def tail_pass():
        def grp(g, _):
            pk = bpack_v[_TAILBK, pl.ds(g * 16, 16)]
            col = pk & 63
            slot = pk >> _CSH
            for f in range(EMBED_DIM):
                v16 = plsc.load_gather(tail_v, [_i16(f), col])
                plsc.store_scatter(ans_v, [slot, _i16(f)], v16)
            return 0

        lax.fori_loop(0, _BCAP // 16, grp, 0)

    tail_pass()

    def scan_pass(p):
        def src(bk):
            off = jnp.minimum(bk * _CW, _SZ31 - _CW)
            return utabT_hbm.at[pl.ds(p * _FH, _FH),
                                pl.ds(gbase + off, _CW)]

        def extract(bk, b):
            adj = jnp.where(_i16(bk) == _i16(15), _i16(1024), _i16(0))

            def grp(g, _):
                pk = bpack_v[bk, pl.ds(g * 16, 16)]
                col = (pk & (_CW - 1)) + adj
                slot = pk >> _CSH
                for f in range(_FH):
                    v16 = plsc.load_gather(chunk_v, [_i16(b), _i16(f), col])
                    plsc.store_scatter(ans_v, [slot, _i16(p * _FH + f)], v16)
                return 0

            lax.fori_loop(0, _BCAP // 16, grp, 0)

        pltpu.async_copy(src(0), chunk_v.at[0], sem)

        @pl.loop(0, _NCHP - 1, step=2)
        def _ring(k):
            pltpu.async_copy(src(k + 1), chunk_v.at[1], sem2)
            pltpu.make_async_copy(src(k), chunk_v.at[0], sem).wait()
            extract(k, 0)
            pltpu.async_copy(src(k + 2), chunk_v.at[0], sem)
            pltpu.make_async_copy(src(k + 1), chunk_v.at[1], sem2).wait()
            extract(k + 1, 1)

        pltpu.make_async_copy(src(_NCHP - 1), chunk_v.at[0], sem).wait()
        extract(_NCHP - 1, 0)

    for p in range(_NPASS):
        team_pass(p)
        scan_pass(p)

    copies = []
    for s in range(_NSTREAM):
        copies.append(pltpu.async_copy(
            ans_v.at[pl.ds(s * 128, 128)], out_hbm.at[qb_v.at[s]], sem))
    for c in copies:
        c.wait()


def kernel(user, favourite_team, user_table, team_table):
    u2 = user.astype(jnp.int32).reshape(128, 128)
    t2 = favourite_team.astype(jnp.int32).reshape(128, 128)
    utabT = user_table.T
    ttabT = jnp.pad(team_table.T, ((0, 0), (0, _CW - NUM_TEAMS)))
    tailT = user_table.T[:, _TAIL_LO:]
    out = _scan_lookup(u2, t2, utabT, ttabT, tailT)
    return out[:BATCH, :2 * EMBED_DIM]


# speedup vs baseline: 1.0146x; 1.0146x over previous
import functools

import jax
import jax.numpy as jnp
from jax import lax
from jax.experimental import pallas as pl
from jax.experimental.pallas import tpu as pltpu
from jax.experimental.pallas import tpu_sc as plsc

NUM_USERS = 1000000
NUM_TEAMS = 1000
EMBED_DIM = 32
BATCH = 16384

_info = plsc.get_sparse_core_info()
_NC, _NS = _info.num_cores, _info.num_subcores
_NW = _NC * _NS
_BPW = BATCH // _NW

_TAIL_LO = (NUM_USERS // 128) * 128
_SZ0 = (_TAIL_LO // 128 // _NW) * 128
_SZ31 = _TAIL_LO - (_NW - 1) * _SZ0
_CW = 1024
_CSH = 10
_FH = 16
_NPASS = EMBED_DIM // _FH
_QCAP = 640
_NSTREAM = _QCAP // 128
_OUT_ROWS = BATCH + 2 * 128
_BCAP = 48
_NCH = _SZ31 // _CW
_TAILBK = _NCH

_mesh = plsc.VectorSubcoreMesh(core_axis_name="c", subcore_axis_name="s")


def _i16(x):
    return jnp.full((16,), x, dtype=jnp.int32)


@functools.partial(
    pl.kernel,
    mesh=_mesh,
    out_type=jax.ShapeDtypeStruct((_OUT_ROWS, 128), jnp.float32),
    compiler_params=pltpu.CompilerParams(needs_layout_passes=False),
    scratch_types=[
        pltpu.VMEM((8, 128), jnp.int32),
        pltpu.VMEM((8, 128), jnp.int32),
        pltpu.VMEM((_QCAP,), jnp.int32),
        pltpu.VMEM((_NSTREAM, 128), jnp.int32),
        pltpu.VMEM((_QCAP,), jnp.int32),
        pltpu.VMEM((2, _FH, _CW), jnp.float32),
        pltpu.VMEM((EMBED_DIM, 64), jnp.float32),
        pltpu.VMEM((_QCAP, 128), jnp.float32),
        pltpu.VMEM((_NCH + 1, _BCAP), jnp.int32),
        pltpu.SemaphoreType.DMA,
        pltpu.SemaphoreType.DMA,
    ],
)
def _scan_lookup(uidx_hbm, tidx_hbm, utabT_hbm, ttabT_hbm, tailT_hbm, out_hbm,
                 up_v, tp_v, qu_v, qb_v, qt_v, chunk_v, tail_v, ans_v,
                 bpack_v, sem, sem2):
    wid = lax.axis_index("s") * _NC + lax.axis_index("c")
    gbase = wid * _SZ0
    lo16 = _i16(gbase)
    hi16 = jnp.where(_i16(wid) == _i16(_NW - 1),
                     _i16(_TAIL_LO), _i16(gbase + _SZ0))
    blo16 = _i16(wid * _BPW)
    bhi16 = _i16(wid * _BPW + _BPW)
    tail16 = _i16(_TAIL_LO)
    iota = lax.iota(jnp.int32, 16)

    def init_q(s, _):
        for h in range(8):
            qb_v[s, pl.ds(h * 16, 16)] = _i16(BATCH + wid * 8) + (iota & 7)
            qu_v[pl.ds(s * 128 + h * 16, 16)] = _i16(-1)
        return 0

    lax.fori_loop(0, _NSTREAM, init_q, 0)

    def piece(p8, base16):
        def group(i, b16c):
            gr = i >> 3
            gc = i & 7
            u16 = up_v[gr, pl.ds(gc * 16, 16)]
            t16 = tp_v[gr, pl.ds(gc * 16, 16)]
            b16 = _i16(p8 * 1024) + _i16(i * 16) + iota
            m_main = (u16 >= lo16) & (u16 < hi16)
            m_tail = (u16 >= tail16) & (b16 >= blo16) & (b16 < bhi16)
            m = m_main | m_tail
            pos = b16c + plsc.cumsum(m.astype(jnp.int32)) - 1
            pos = jnp.minimum(pos, _QCAP - 2)
            plsc.store_scatter(qu_v, [pos], u16, mask=m)
            plsc.store_scatter(qt_v, [pos], t16, mask=m)
            plsc.store_scatter(qb_v, [pos >> 7, pos & 127], b16, mask=m)
            return b16c + plsc.all_reduce_population_count(m)

        pltpu.sync_copy(uidx_hbm.at[pl.ds(p8 * 8, 8)], up_v)
        pltpu.sync_copy(tidx_hbm.at[pl.ds(p8 * 8, 8)], tp_v)
        return lax.fori_loop(0, 64, group, base16)

    @pl.loop(0, 16, init_carry=_i16(0))
    def base16(p8, carry):
        return piece(p8, carry)

    def init_b(bk, _):
        for h in range(_BCAP // 16):
            bpack_v[bk, pl.ds(h * 16, 16)] = _i16((_QCAP - 1) << _CSH)
        return 0

    lax.fori_loop(0, _NCH + 1, init_b, 0)

    @pl.loop(0, _NCH)
    def _bucket(bk):
        def bkt(qg, cnt16):
            u16 = qu_v[pl.ds(qg * 16, 16)]
            rel = u16 - lo16
            m = ((rel >> _CSH) == _i16(bk)) & (u16 >= lo16) & (u16 < hi16)
            pos = jnp.minimum(cnt16 + plsc.cumsum(m.astype(jnp.int32)) - 1,
                              _BCAP - 1)
            slot = _i16(qg * 16) + iota
            plsc.store_scatter(bpack_v, [_i16(bk), pos],
                               (rel - _i16(bk * _CW)) | (slot << _CSH), mask=m)
            return cnt16 + plsc.all_reduce_population_count(m)

        lax.fori_loop(0, _QCAP // 16, bkt, _i16(0))

    def tailbkt(qg, cnt16):
        u16 = qu_v[pl.ds(qg * 16, 16)]
        m = u16 >= tail16
        pos = jnp.minimum(cnt16 + plsc.cumsum(m.astype(jnp.int32)) - 1,
                          _BCAP - 1)
        slot = _i16(qg * 16) + iota
        plsc.store_scatter(bpack_v, [_i16(_TAILBK), pos],
                           (u16 - tail16) | (slot << _CSH), mask=m)
        return cnt16 + plsc.all_reduce_population_count(m)

    lax.fori_loop(0, _QCAP // 16, tailbkt, _i16(0))

    pltpu.sync_copy(tailT_hbm, tail_v)

    def team_pass(p):
        def grp(qg, _):
            tcol = jnp.clip(qt_v[pl.ds(qg * 16, 16)], 0, _CW - 1)
            slot = _i16(qg * 16) + iota
            for f in range(_FH):
                v16 = plsc.load_gather(chunk_v, [_i16(0), _i16(f), tcol])
                plsc.store_scatter(ans_v, [slot, _i16(EMBED_DIM + p * _FH + f)], v16)
            return 0

        pltpu.sync_copy(ttabT_hbm.at[pl.ds(p * _FH, _FH)], chunk_v.at[0])
        lax.fori_loop(0, _QCAP // 16, grp, 0)

    def tail_pass():
        def grp(g, _):
            pk = bpack_v[_TAILBK, pl.ds(g * 16, 16)]
            col = pk & 63
            slot = pk >> _CSH
            for f in range(EMBED_DIM):
                v16 = plsc.load_gather(tail_v, [_i16(f), col])
                plsc.store_scatter(ans_v, [slot, _i16(f)], v16)
            return 0

        lax.fori_loop(0, _BCAP // 16, grp, 0)

    tail_pass()

    def scan_pass(p):
        def src(bk):
            return utabT_hbm.at[pl.ds(p * _FH, _FH),
                                pl.ds(gbase + bk * _CW, _CW)]

        def extract(bk, b):
            def grp(g, _):
                pk = bpack_v[bk, pl.ds(g * 16, 16)]
                col = pk & (_CW - 1)
                slot = pk >> _CSH
                for f in range(_FH):
                    v16 = plsc.load_gather(chunk_v, [_i16(b), _i16(f), col])
                    plsc.store_scatter(ans_v, [slot, _i16(p * _FH + f)], v16)
                return 0

            lax.fori_loop(0, _BCAP // 16, grp, 0)

        pltpu.async_copy(src(0), chunk_v.at[0], sem)

        @pl.loop(0, _NCH - 1, step=2)
        def _ring(k):
            pltpu.async_copy(src(k + 1), chunk_v.at[1], sem2)
            pltpu.make_async_copy(src(k), chunk_v.at[0], sem).wait()
            extract(k, 0)
            pltpu.async_copy(src(k + 2), chunk_v.at[0], sem)
            pltpu.make_async_copy(src(k + 1), chunk_v.at[1], sem2).wait()
            extract(k + 1, 1)

        pltpu.make_async_copy(src(_NCH - 1), chunk_v.at[0], sem).wait()
        extract(_NCH - 1, 0)

    for p in range(_NPASS):
        team_pass(p)
        scan_pass(p)

    copies = []
    for s in range(_NSTREAM):
        copies.append(pltpu.async_copy(
            ans_v.at[pl.ds(s * 128, 128)], out_hbm.at[qb_v.at[s]], sem))
    for c in copies:
        c.wait()


def kernel(user, favourite_team, user_table, team_table):
    u2 = user.astype(jnp.int32).reshape(128, 128)
    t2 = favourite_team.astype(jnp.int32).reshape(128, 128)
    utabT = user_table.T
    ttabT = jnp.pad(team_table.T, ((0, 0), (0, _CW - NUM_TEAMS)))
    tailT = user_table.T[:, _TAIL_LO:]
    out = _scan_lookup(u2, t2, utabT, ttabT, tailT)
    return out[:BATCH, :2 * EMBED_DIM]
